# baseline (speedup 1.0000x reference)
import functools

import jax
import jax.numpy as jnp
from jax import lax
from jax.experimental import pallas as pl
from jax.experimental.pallas import tpu as pltpu
from jax.experimental.pallas import tpu_sc as plsc

B, F, V, D, O = 4096, 26, 1000, 128, 256

NC, NS, L = 2, 16, 16
NW = NC * NS
ROWS_PER_W = B // NW
CB = 4
NCHUNK = ROWS_PER_W // CB
IDX_PER_CHUNK = CB * F
OC = O // L
NBUF = 4


FPB = 26


def _pmm_body(table_ref, w_ref, p_ref):
    for k in range(FPB):
        y = jnp.dot(table_ref[...], w_ref[pl.ds(k * D, D), :],
                    preferred_element_type=jnp.float32)
        lo = lax.bitcast_convert_type(y[:, :O // 2].astype(jnp.bfloat16),
                                      jnp.uint16).astype(jnp.uint32)
        hi = lax.bitcast_convert_type(y[:, O // 2:].astype(jnp.bfloat16),
                                      jnp.uint16).astype(jnp.uint32)
        p_ref[pl.ds(k * V, V), :] = lax.bitcast_convert_type(lo | (hi << 16),
                                                             jnp.int32)


def _precompute_p(table, w):
    return pl.pallas_call(
        _pmm_body,
        grid=(F // FPB,),
        in_specs=[
            pl.BlockSpec((V, D), lambda f: (0, 0)),
            pl.BlockSpec((FPB * D, O), lambda f: (f, 0)),
        ],
        out_specs=pl.BlockSpec((FPB * V, O // 2), lambda f: (f, 0)),
        out_shape=jax.ShapeDtypeStruct((F * V, O // 2), jnp.int32),
    )(table, w)


def _sc_bag_body(p_hbm, g_hbm, out_hbm, idx_v, rows_v, acc_v, sem_g, sem_o):
    wid = lax.axis_index("s") * NC + lax.axis_index("c")
    base = wid * ROWS_PER_W

    pltpu.sync_copy(g_hbm.at[pl.ds(base * F, ROWS_PER_W * F)], idx_v)

    def gather_desc(i, par):
        off = pl.multiple_of(i * IDX_PER_CHUNK, 8)
        return pltpu.make_async_copy(
            p_hbm.at[idx_v.at[pl.ds(off, IDX_PER_CHUNK)]],
            rows_v.at[par], sem_g[par])

    def out_desc(i, par):
        return pltpu.make_async_copy(
            acc_v.at[par], out_hbm.at[pl.ds(base + i * CB, CB)], sem_o[par])

    for par in range(NBUF):
        gather_desc(par, par).start()

    def chunk(i, par):
        gather_desc(i, par).wait()
        @pl.when(i >= NBUF)
        def _():
            out_desc(i - NBUF, par).wait()
        for r in range(CB):
            def unpacked(f):
                los, his = [], []
                for c in range(OC // 2):
                    w = rows_v[par, r * F + f, pl.ds(c * L, L)]
                    los.append(lax.bitcast_convert_type(w << 16, jnp.float32))
                    his.append(lax.bitcast_convert_type(w, jnp.float32))
                return los, his

            def fbody(f, acc):
                lo, hi = unpacked(f)
                return tuple(acc[c] + lo[c] for c in range(OC // 2)) + \
                       tuple(acc[OC // 2 + c] + hi[c] for c in range(OC // 2))
            lo, hi = unpacked(0)
            acc = lax.fori_loop(1, F, fbody, tuple(lo) + tuple(hi))
            for c in range(OC):
                acc_v[par, r, pl.ds(c * L, L)] = acc[c]
        @pl.when(i + NBUF < NCHUNK)
        def _():
            gather_desc(i + NBUF, par).start()
        out_desc(i, par).start()

    def ring(j, carry):
        for par in range(NBUF):
            chunk(NBUF * j + par, par)
        return carry

    lax.fori_loop(0, NCHUNK // NBUF, ring, 0)
    for par in range(NBUF):
        out_desc(NCHUNK - NBUF + par, par).wait()


@functools.cache
def _sc_bag():
    return pl.kernel(
        _sc_bag_body,
        out_type=jax.ShapeDtypeStruct((B, O), jnp.float32),
        mesh=plsc.VectorSubcoreMesh(core_axis_name="c", subcore_axis_name="s",
                                    num_cores=NC, num_subcores=NS),
        scratch_types=[
            pltpu.VMEM((ROWS_PER_W * F,), jnp.int32),
            pltpu.VMEM((NBUF, IDX_PER_CHUNK, O // 2), jnp.int32),
            pltpu.VMEM((NBUF, CB, O), jnp.float32),
            [pltpu.SemaphoreType.DMA] * NBUF,
            [pltpu.SemaphoreType.DMA] * NBUF,
        ],
    )


LN_BLK = 2048


def _ln_body(y_ref, b_ref, g_ref, be_ref, o_ref):
    y = y_ref[...] + b_ref[...]
    mean = jnp.mean(y, axis=-1, keepdims=True)
    yc = y - mean
    var = jnp.mean(yc * yc, axis=-1, keepdims=True)
    ln = yc * lax.rsqrt(var + 1e-5) * g_ref[...] + be_ref[...]
    o_ref[...] = ln * jax.nn.sigmoid(ln)


def _ln_silu(ysum, b, gamma, beta):
    return pl.pallas_call(
        _ln_body,
        grid=(B // LN_BLK,),
        in_specs=[
            pl.BlockSpec((LN_BLK, O), lambda i: (i, 0)),
            pl.BlockSpec((1, O), lambda i: (0, 0)),
            pl.BlockSpec((1, O), lambda i: (0, 0)),
            pl.BlockSpec((1, O), lambda i: (0, 0)),
        ],
        out_specs=pl.BlockSpec((LN_BLK, O), lambda i: (i, 0)),
        out_shape=jax.ShapeDtypeStruct((B, O), jnp.float32),
    )(ysum, b.reshape(1, O), gamma.reshape(1, O), beta.reshape(1, O))


@jax.jit
def kernel(x, low, table, W, b, gamma, beta):
    g = ((x.astype(jnp.int32) - low)
         + (jnp.arange(F, dtype=jnp.int32) * V)[None, :]).reshape(-1)
    p = _precompute_p(table, W)
    ysum = _sc_bag()(p, g)
    return _ln_silu(ysum, b, gamma, beta)


# speedup vs baseline: 1.0346x; 1.0346x over previous
import functools

import jax
import jax.numpy as jnp
from jax import lax
from jax.experimental import pallas as pl
from jax.experimental.pallas import tpu as pltpu
from jax.experimental.pallas import tpu_sc as plsc

B, F, V, D, O = 4096, 26, 1000, 128, 256

NC, NS, L = 2, 16, 16
NW = NC * NS
ROWS_PER_W = B // NW
CB = 4
NCHUNK = ROWS_PER_W // CB
IDX_PER_CHUNK = CB * F
OC = O // L
NBUF = 4


FPB = 13


def _pmm_body(table_ref, w_ref, p_ref):
    for k in range(FPB):
        y = jnp.dot(table_ref[...], w_ref[pl.ds(k * D, D), :],
                    preferred_element_type=jnp.float32)
        lo = lax.bitcast_convert_type(y[:, :O // 2].astype(jnp.bfloat16),
                                      jnp.uint16).astype(jnp.uint32)
        hi = lax.bitcast_convert_type(y[:, O // 2:].astype(jnp.bfloat16),
                                      jnp.uint16).astype(jnp.uint32)
        p_ref[pl.ds(k * V, V), :] = lax.bitcast_convert_type(lo | (hi << 16),
                                                             jnp.int32)


def _precompute_p(table, w):
    return pl.pallas_call(
        _pmm_body,
        grid=(F // FPB,),
        in_specs=[
            pl.BlockSpec((V, D), lambda f: (0, 0)),
            pl.BlockSpec((FPB * D, O), lambda f: (f, 0)),
        ],
        out_specs=pl.BlockSpec((FPB * V, O // 2), lambda f: (f, 0)),
        out_shape=jax.ShapeDtypeStruct((F * V, O // 2), jnp.int32),
    )(table, w)


def _sc_bag_body(p_hbm, g_hbm, out_hbm, idx_v, rows_v, acc_v, sem_g, sem_o):
    wid = lax.axis_index("s") * NC + lax.axis_index("c")
    base = wid * ROWS_PER_W

    pltpu.sync_copy(g_hbm.at[pl.ds(base * F, ROWS_PER_W * F)], idx_v)

    def gather_desc(i, par):
        off = pl.multiple_of(i * IDX_PER_CHUNK, 8)
        return pltpu.make_async_copy(
            p_hbm.at[idx_v.at[pl.ds(off, IDX_PER_CHUNK)]],
            rows_v.at[par], sem_g[par])

    def out_desc(i, par):
        return pltpu.make_async_copy(
            acc_v.at[par], out_hbm.at[pl.ds(base + i * CB, CB)], sem_o[par])

    for par in range(NBUF):
        gather_desc(par, par).start()

    def chunk(i, par):
        gather_desc(i, par).wait()
        @pl.when(i >= NBUF)
        def _():
            out_desc(i - NBUF, par).wait()
        for r in range(CB):
            def unpacked(f):
                los, his = [], []
                for c in range(OC // 2):
                    w = rows_v[par, r * F + f, pl.ds(c * L, L)]
                    los.append(lax.bitcast_convert_type(w << 16, jnp.float32))
                    his.append(lax.bitcast_convert_type(w, jnp.float32))
                return los, his

            def fbody(f, acc):
                lo, hi = unpacked(f)
                return tuple(acc[c] + lo[c] for c in range(OC // 2)) + \
                       tuple(acc[OC // 2 + c] + hi[c] for c in range(OC // 2))
            lo, hi = unpacked(0)
            acc = lax.fori_loop(1, F, fbody, tuple(lo) + tuple(hi))
            for c in range(OC):
                acc_v[par, r, pl.ds(c * L, L)] = acc[c]
        @pl.when(i + NBUF < NCHUNK)
        def _():
            gather_desc(i + NBUF, par).start()
        out_desc(i, par).start()

    def ring(j, carry):
        for par in range(NBUF):
            chunk(NBUF * j + par, par)
        return carry

    lax.fori_loop(0, NCHUNK // NBUF, ring, 0)
    for par in range(NBUF):
        out_desc(NCHUNK - NBUF + par, par).wait()


@functools.cache
def _sc_bag():
    return pl.kernel(
        _sc_bag_body,
        out_type=jax.ShapeDtypeStruct((B, O), jnp.float32),
        mesh=plsc.VectorSubcoreMesh(core_axis_name="c", subcore_axis_name="s",
                                    num_cores=NC, num_subcores=NS),
        scratch_types=[
            pltpu.VMEM((ROWS_PER_W * F,), jnp.int32),
            pltpu.VMEM((NBUF, IDX_PER_CHUNK, O // 2), jnp.int32),
            pltpu.VMEM((NBUF, CB, O), jnp.float32),
            [pltpu.SemaphoreType.DMA] * NBUF,
            [pltpu.SemaphoreType.DMA] * NBUF,
        ],
    )


LN_BLK = 2048


def _ln_body(y_ref, b_ref, g_ref, be_ref, o_ref):
    y = y_ref[...] + b_ref[...]
    mean = jnp.mean(y, axis=-1, keepdims=True)
    yc = y - mean
    var = jnp.mean(yc * yc, axis=-1, keepdims=True)
    ln = yc * lax.rsqrt(var + 1e-5) * g_ref[...] + be_ref[...]
    o_ref[...] = ln * jax.nn.sigmoid(ln)


def _ln_silu(ysum, b, gamma, beta):
    return pl.pallas_call(
        _ln_body,
        grid=(B // LN_BLK,),
        in_specs=[
            pl.BlockSpec((LN_BLK, O), lambda i: (i, 0)),
            pl.BlockSpec((1, O), lambda i: (0, 0)),
            pl.BlockSpec((1, O), lambda i: (0, 0)),
            pl.BlockSpec((1, O), lambda i: (0, 0)),
        ],
        out_specs=pl.BlockSpec((LN_BLK, O), lambda i: (i, 0)),
        out_shape=jax.ShapeDtypeStruct((B, O), jnp.float32),
    )(ysum, b.reshape(1, O), gamma.reshape(1, O), beta.reshape(1, O))


@jax.jit
def kernel(x, low, table, W, b, gamma, beta):
    g = ((x.astype(jnp.int32) - low)
         + (jnp.arange(F, dtype=jnp.int32) * V)[None, :]).reshape(-1)
    p = _precompute_p(table, W)
    ysum = _sc_bag()(p, g)
    return _ln_silu(ysum, b, gamma, beta)
